# baseline (speedup 1.0000x reference)
import functools

import jax
import jax.numpy as jnp
from jax import lax
from jax.experimental import pallas as pl
from jax.experimental.pallas import tpu as pltpu
from jax.experimental.pallas import tpu_sc as plsc

_P = 512
_LANES = 128


def _fps_body(x_ref, y_ref, z_ref, d0_ref, f0_ref,
              inds_ref, sx_ref, sy_ref, sz_ref, d_ref):
    B, Np = x_ref.shape
    T = Np // _LANES
    d_ref[...] = d0_ref[...]
    lane8 = lax.broadcasted_iota(jnp.int32, (B, _LANES), 1)
    lane_full = lax.broadcasted_iota(jnp.int32, (B, Np), 1)

    far0 = f0_ref[...]
    eq = lane_full == far0
    zero_full = jnp.zeros((B, Np), jnp.float32)
    cx0 = jnp.sum(jnp.where(eq, x_ref[...], zero_full), axis=1, keepdims=True)
    cy0 = jnp.sum(jnp.where(eq, y_ref[...], zero_full), axis=1, keepdims=True)
    cz0 = jnp.sum(jnp.where(eq, z_ref[...], zero_full), axis=1, keepdims=True)

    zero128 = jnp.zeros((B, _LANES), jnp.float32)

    def it(i, carry):
        far, cx, cy, cz = carry
        col0 = pl.multiple_of((i // _LANES) * _LANES, _LANES)
        msk = lane8 == (i % _LANES)
        oc = pl.ds(col0, _LANES)
        inds_ref[:, oc] = jnp.where(msk, far, inds_ref[:, oc])
        sx_ref[:, oc] = jnp.where(msk, cx, sx_ref[:, oc])
        sy_ref[:, oc] = jnp.where(msk, cy, sy_ref[:, oc])
        sz_ref[:, oc] = jnp.where(msk, cz, sz_ref[:, oc])

        NA = 4
        run_v = [jnp.full((B, _LANES), -jnp.inf, jnp.float32)] * NA
        run_t = [jnp.zeros((B, _LANES), jnp.int32)] * NA
        run_x = [zero128] * NA
        run_y = [zero128] * NA
        run_z = [zero128] * NA
        for t in range(T):
            j = t % NA
            s = pl.ds(t * _LANES, _LANES)
            xv = x_ref[:, s]
            yv = y_ref[:, s]
            zv = z_ref[:, s]
            dx = xv - cx
            dy = yv - cy
            dz = zv - cz
            dist = dx * dx + dy * dy + dz * dz
            nd = jnp.minimum(d_ref[:, s], dist)
            d_ref[:, s] = nd
            cond = nd > run_v[j]
            run_v[j] = jnp.maximum(run_v[j], nd)
            run_t[j] = jnp.where(cond, t, run_t[j])
            run_x[j] = jnp.where(cond, xv, run_x[j])
            run_y[j] = jnp.where(cond, yv, run_y[j])
            run_z[j] = jnp.where(cond, zv, run_z[j])

        def merge(a, b):
            av, at, ax, ay, az = a
            bv, bt, bx, by, bz = b
            take_b = (bv > av) | ((bv == av) & (bt < at))
            return (jnp.where(take_b, bv, av), jnp.where(take_b, bt, at),
                    jnp.where(take_b, bx, ax), jnp.where(take_b, by, ay),
                    jnp.where(take_b, bz, az))

        sets = [(run_v[j], run_t[j], run_x[j], run_y[j], run_z[j])
                for j in range(NA)]
        mv, mt, mx, my, mz = merge(merge(sets[0], sets[1]),
                                   merge(sets[2], sets[3]))
        run_v, run_t, run_x, run_y, run_z = mv, mt, mx, my, mz

        gidx = run_t * _LANES + lane8
        m = jnp.max(run_v, axis=1, keepdims=True)
        ksel = jnp.where(run_v == m, gidx, Np)
        nf = jnp.min(ksel, axis=1, keepdims=True)
        fm = ksel == nf
        stacked = jnp.concatenate(
            [jnp.where(fm, run_x, zero128),
             jnp.where(fm, run_y, zero128),
             jnp.where(fm, run_z, zero128)], axis=0)
        csum = jnp.sum(stacked, axis=1, keepdims=True)
        return (nf, csum[:B], csum[B:2 * B], csum[2 * B:])

    lax.fori_loop(0, _P, it, (far0, cx0, cy0, cz0))


def _fps_tc(x, y, z, d0, f0):
    B, Np = x.shape
    return pl.pallas_call(
        _fps_body,
        out_shape=[
            jax.ShapeDtypeStruct((B, _P), jnp.int32),
            jax.ShapeDtypeStruct((B, _P), jnp.float32),
            jax.ShapeDtypeStruct((B, _P), jnp.float32),
            jax.ShapeDtypeStruct((B, _P), jnp.float32),
        ],
        scratch_shapes=[pltpu.VMEM((B, Np), jnp.float32)],
    )(x, y, z, d0, f0)


def _gather_sc(feat2d, inds):
    R, N = feat2d.shape
    B = inds.shape[0]
    C = R // B
    NW = 32
    rpw = R // NW
    mesh = plsc.VectorSubcoreMesh(core_axis_name="c", subcore_axis_name="s")

    @functools.partial(
        pl.kernel,
        out_type=jax.ShapeDtypeStruct((R, _P), jnp.float32),
        mesh=mesh,
        compiler_params=pltpu.CompilerParams(needs_layout_passes=False),
        scratch_types=[
            pltpu.VMEM((_P,), jnp.int32),
            pltpu.VMEM((2, N), jnp.float32),
            pltpu.VMEM((2, N), jnp.float32),
            pltpu.VMEM((rpw, _P), jnp.float32),
            pltpu.SemaphoreType.DMA,
            pltpu.SemaphoreType.DMA,
        ],
    )
    def gather_k(feat_hbm, idx_hbm, out_hbm, idx_v, row_a, row_b,
                 out_acc, sem_a, sem_b):
        cid = lax.axis_index("c")
        sid = lax.axis_index("s")
        wid = sid * 2 + cid
        row0 = wid * rpw
        b = row0 // C
        pltpu.sync_copy(idx_hbm.at[b], idx_v)
        pltpu.async_copy(feat_hbm.at[pl.ds(row0, 2)], row_a, sem_a)

        def pick(buf_ref, j, r):
            for k in range(_P // 16):
                iv = idx_v[pl.ds(k * 16, 16)]
                jv = jnp.full((16,), j, jnp.int32)
                out_acc[r, pl.ds(k * 16, 16)] = plsc.load_gather(
                    buf_ref, [jv, iv])

        def chunk_body(k, _):
            r0 = 4 * k
            pltpu.make_async_copy(feat_hbm.at[pl.ds(row0, 2)],
                                  row_a, sem_a).wait()
            pltpu.async_copy(feat_hbm.at[pl.ds(row0 + r0 + 2, 2)],
                             row_b, sem_b)
            pick(row_a, 0, r0)
            pick(row_a, 1, r0 + 1)
            nxt = jnp.minimum(r0 + 4, rpw - 2)
            pltpu.make_async_copy(feat_hbm.at[pl.ds(row0, 2)],
                                  row_b, sem_b).wait()
            pltpu.async_copy(feat_hbm.at[pl.ds(row0 + nxt, 2)],
                             row_a, sem_a)
            pick(row_b, 0, r0 + 2)
            pick(row_b, 1, r0 + 3)
            return 0

        lax.fori_loop(0, rpw // 4, chunk_body, 0)
        pltpu.make_async_copy(feat_hbm.at[pl.ds(row0, 2)],
                              row_a, sem_a).wait()
        pltpu.sync_copy(out_acc, out_hbm.at[pl.ds(row0, rpw)])

    return gather_k(feat2d, inds)


def kernel(xyz, features):
    B, N, _ = xyz.shape
    C = features.shape[1]
    Np = ((N + _LANES - 1) // _LANES) * _LANES

    feat2d = features.reshape(B * C, N)

    pad = Np - N
    x = jnp.pad(xyz[:, :, 0], ((0, 0), (0, pad)))
    y = jnp.pad(xyz[:, :, 1], ((0, 0), (0, pad)))
    z = jnp.pad(xyz[:, :, 2], ((0, 0), (0, pad)))
    d0 = jnp.concatenate(
        [jnp.full((B, N), 1e10, jnp.float32),
         jnp.full((B, pad), -jnp.inf, jnp.float32)], axis=1)
    f0 = jax.random.randint(jax.random.key(1), (B,), 0, N,
                            dtype=jnp.int32)[:, None]

    inds, sx, sy, sz = _fps_tc(x, y, z, d0, f0)
    new_xyz = jnp.stack([sx, sy, sz], axis=-1)

    new_features = _gather_sc(feat2d, inds).reshape(B, C, _P)
    return (new_xyz, new_features, inds)


# speedup vs baseline: 1.0770x; 1.0770x over previous
import functools

import jax
import jax.numpy as jnp
from jax import lax
from jax.experimental import pallas as pl
from jax.experimental.pallas import tpu as pltpu
from jax.experimental.pallas import tpu_sc as plsc

_P = 512
_LANES = 128


def _fps_body(x_ref, y_ref, z_ref, d0_ref, f0_ref,
              inds_ref, sx_ref, sy_ref, sz_ref, d_ref):
    B, Np = x_ref.shape
    T = Np // _LANES
    d_ref[...] = d0_ref[...]
    lane8 = lax.broadcasted_iota(jnp.int32, (B, _LANES), 1)
    lane_full = lax.broadcasted_iota(jnp.int32, (B, Np), 1)

    far0 = f0_ref[...]
    eq = lane_full == far0
    zero_full = jnp.zeros((B, Np), jnp.float32)
    cx0 = jnp.sum(jnp.where(eq, x_ref[...], zero_full), axis=1, keepdims=True)
    cy0 = jnp.sum(jnp.where(eq, y_ref[...], zero_full), axis=1, keepdims=True)
    cz0 = jnp.sum(jnp.where(eq, z_ref[...], zero_full), axis=1, keepdims=True)

    zero128 = jnp.zeros((B, _LANES), jnp.float32)

    def it(i, carry):
        far, cx, cy, cz = carry
        col0 = pl.multiple_of((i // _LANES) * _LANES, _LANES)
        msk = lane8 == (i % _LANES)
        oc = pl.ds(col0, _LANES)
        inds_ref[:, oc] = jnp.where(msk, far, inds_ref[:, oc])
        sx_ref[:, oc] = jnp.where(msk, cx, sx_ref[:, oc])
        sy_ref[:, oc] = jnp.where(msk, cy, sy_ref[:, oc])
        sz_ref[:, oc] = jnp.where(msk, cz, sz_ref[:, oc])

        run_v = jnp.full((B, _LANES), -jnp.inf, jnp.float32)
        run_t = jnp.zeros((B, _LANES), jnp.int32)
        run_x = zero128
        run_y = zero128
        run_z = zero128
        for t in range(T):
            s = pl.ds(t * _LANES, _LANES)
            xv = x_ref[:, s]
            yv = y_ref[:, s]
            zv = z_ref[:, s]
            dx = xv - cx
            dy = yv - cy
            dz = zv - cz
            dist = dx * dx + dy * dy + dz * dz
            nd = jnp.minimum(d_ref[:, s], dist)
            d_ref[:, s] = nd
            cond = nd > run_v
            run_v = jnp.maximum(run_v, nd)
            run_t = jnp.where(cond, t, run_t)
            run_x = jnp.where(cond, xv, run_x)
            run_y = jnp.where(cond, yv, run_y)
            run_z = jnp.where(cond, zv, run_z)

        gidx = run_t * _LANES + lane8
        m = jnp.max(run_v, axis=1, keepdims=True)
        ksel = jnp.where(run_v == m, gidx, Np)
        nf = jnp.min(ksel, axis=1, keepdims=True)
        fm = ksel == nf
        stacked = jnp.concatenate(
            [jnp.where(fm, run_x, zero128),
             jnp.where(fm, run_y, zero128),
             jnp.where(fm, run_z, zero128)], axis=0)
        csum = jnp.sum(stacked, axis=1, keepdims=True)
        return (nf, csum[:B], csum[B:2 * B], csum[2 * B:])

    lax.fori_loop(0, _P, it, (far0, cx0, cy0, cz0))


def _fps_tc(x, y, z, d0, f0):
    B, Np = x.shape
    return pl.pallas_call(
        _fps_body,
        out_shape=[
            jax.ShapeDtypeStruct((B, _P), jnp.int32),
            jax.ShapeDtypeStruct((B, _P), jnp.float32),
            jax.ShapeDtypeStruct((B, _P), jnp.float32),
            jax.ShapeDtypeStruct((B, _P), jnp.float32),
        ],
        scratch_shapes=[pltpu.VMEM((B, Np), jnp.float32)],
    )(x, y, z, d0, f0)


def _gather_sc(feat2d, inds):
    R, N = feat2d.shape
    B = inds.shape[0]
    C = R // B
    NW = 32
    rpw = R // NW
    mesh = plsc.VectorSubcoreMesh(core_axis_name="c", subcore_axis_name="s")

    @functools.partial(
        pl.kernel,
        out_type=jax.ShapeDtypeStruct((R, _P), jnp.float32),
        mesh=mesh,
        compiler_params=pltpu.CompilerParams(needs_layout_passes=False),
        scratch_types=[
            pltpu.VMEM((_P,), jnp.int32),
            pltpu.VMEM((2, N), jnp.float32),
            pltpu.VMEM((2, N), jnp.float32),
            pltpu.VMEM((rpw, _P), jnp.float32),
            pltpu.SemaphoreType.DMA,
            pltpu.SemaphoreType.DMA,
        ],
    )
    def gather_k(feat_hbm, idx_hbm, out_hbm, idx_v, row_a, row_b,
                 out_acc, sem_a, sem_b):
        cid = lax.axis_index("c")
        sid = lax.axis_index("s")
        wid = sid * 2 + cid
        row0 = wid * rpw
        b = row0 // C
        pltpu.sync_copy(idx_hbm.at[b], idx_v)
        pltpu.async_copy(feat_hbm.at[pl.ds(row0, 2)], row_a, sem_a)

        def pick(buf_ref, j, r):
            for k in range(_P // 16):
                iv = idx_v[pl.ds(k * 16, 16)]
                jv = jnp.full((16,), j, jnp.int32)
                out_acc[r, pl.ds(k * 16, 16)] = plsc.load_gather(
                    buf_ref, [jv, iv])

        def chunk_body(k, _):
            r0 = 4 * k
            pltpu.make_async_copy(feat_hbm.at[pl.ds(row0, 2)],
                                  row_a, sem_a).wait()
            pltpu.async_copy(feat_hbm.at[pl.ds(row0 + r0 + 2, 2)],
                             row_b, sem_b)
            pick(row_a, 0, r0)
            pick(row_a, 1, r0 + 1)
            nxt = jnp.minimum(r0 + 4, rpw - 2)
            pltpu.make_async_copy(feat_hbm.at[pl.ds(row0, 2)],
                                  row_b, sem_b).wait()
            pltpu.async_copy(feat_hbm.at[pl.ds(row0 + nxt, 2)],
                             row_a, sem_a)
            pick(row_b, 0, r0 + 2)
            pick(row_b, 1, r0 + 3)
            return 0

        lax.fori_loop(0, rpw // 4, chunk_body, 0)
        pltpu.make_async_copy(feat_hbm.at[pl.ds(row0, 2)],
                              row_a, sem_a).wait()
        pltpu.sync_copy(out_acc, out_hbm.at[pl.ds(row0, rpw)])

    return gather_k(feat2d, inds)


def kernel(xyz, features):
    B, N, _ = xyz.shape
    C = features.shape[1]
    Np = ((N + _LANES - 1) // _LANES) * _LANES

    feat2d = features.reshape(B * C, N)

    pad = Np - N
    x = jnp.pad(xyz[:, :, 0], ((0, 0), (0, pad)))
    y = jnp.pad(xyz[:, :, 1], ((0, 0), (0, pad)))
    z = jnp.pad(xyz[:, :, 2], ((0, 0), (0, pad)))
    d0 = jnp.concatenate(
        [jnp.full((B, N), 1e10, jnp.float32),
         jnp.full((B, pad), -jnp.inf, jnp.float32)], axis=1)
    f0 = jax.random.randint(jax.random.key(1), (B,), 0, N,
                            dtype=jnp.int32)[:, None]

    inds, sx, sy, sz = _fps_tc(x, y, z, d0, f0)
    new_xyz = jnp.stack([sx, sy, sz], axis=-1)

    new_features = _gather_sc(feat2d, inds).reshape(B, C, _P)
    return (new_xyz, new_features, inds)


# speedup vs baseline: 1.0883x; 1.0105x over previous
import functools

import jax
import jax.numpy as jnp
from jax import lax
from jax.experimental import pallas as pl
from jax.experimental.pallas import tpu as pltpu
from jax.experimental.pallas import tpu_sc as plsc

_P = 512
_LANES = 128


def _fps_body(n_real, x_ref, y_ref, z_ref, f0_ref,
              inds_ref, sx_ref, sy_ref, sz_ref, d_ref):
    B, Np = x_ref.shape
    T = Np // _LANES
    lane8 = lax.broadcasted_iota(jnp.int32, (B, _LANES), 1)
    lane_full = lax.broadcasted_iota(jnp.int32, (B, Np), 1)
    d_ref[...] = jnp.where(lane_full < n_real,
                           jnp.float32(1e10), -jnp.inf)

    far0 = f0_ref[...]
    eq = lane_full == far0
    zero_full = jnp.zeros((B, Np), jnp.float32)
    cx0 = jnp.sum(jnp.where(eq, x_ref[...], zero_full), axis=1, keepdims=True)
    cy0 = jnp.sum(jnp.where(eq, y_ref[...], zero_full), axis=1, keepdims=True)
    cz0 = jnp.sum(jnp.where(eq, z_ref[...], zero_full), axis=1, keepdims=True)

    zero128 = jnp.zeros((B, _LANES), jnp.float32)

    def it(i, carry):
        far, cx, cy, cz = carry
        col0 = pl.multiple_of((i // _LANES) * _LANES, _LANES)
        msk = lane8 == (i % _LANES)
        oc = pl.ds(col0, _LANES)
        inds_ref[:, oc] = jnp.where(msk, far, inds_ref[:, oc])
        sx_ref[:, oc] = jnp.where(msk, cx, sx_ref[:, oc])
        sy_ref[:, oc] = jnp.where(msk, cy, sy_ref[:, oc])
        sz_ref[:, oc] = jnp.where(msk, cz, sz_ref[:, oc])

        run_v = jnp.full((B, _LANES), -jnp.inf, jnp.float32)
        run_t = jnp.zeros((B, _LANES), jnp.int32)
        run_x = zero128
        run_y = zero128
        run_z = zero128
        for t in range(T):
            s = pl.ds(t * _LANES, _LANES)
            xv = x_ref[:, s]
            yv = y_ref[:, s]
            zv = z_ref[:, s]
            dx = xv - cx
            dy = yv - cy
            dz = zv - cz
            dist = dx * dx + dy * dy + dz * dz
            nd = jnp.minimum(d_ref[:, s], dist)
            d_ref[:, s] = nd
            cond = nd > run_v
            run_v = jnp.maximum(run_v, nd)
            run_t = jnp.where(cond, t, run_t)
            run_x = jnp.where(cond, xv, run_x)
            run_y = jnp.where(cond, yv, run_y)
            run_z = jnp.where(cond, zv, run_z)

        gidx = run_t * _LANES + lane8
        m = jnp.max(run_v, axis=1, keepdims=True)
        ksel = jnp.where(run_v == m, gidx, Np)
        nf = jnp.min(ksel, axis=1, keepdims=True)
        fm = ksel == nf
        stacked = jnp.concatenate(
            [jnp.where(fm, run_x, zero128),
             jnp.where(fm, run_y, zero128),
             jnp.where(fm, run_z, zero128)], axis=0)
        csum = jnp.sum(stacked, axis=1, keepdims=True)
        return (nf, csum[:B], csum[B:2 * B], csum[2 * B:])

    lax.fori_loop(0, _P, it, (far0, cx0, cy0, cz0))


def _fps_tc(x, y, z, n_real, f0):
    B, Np = x.shape
    return pl.pallas_call(
        functools.partial(_fps_body, n_real),
        out_shape=[
            jax.ShapeDtypeStruct((B, _P), jnp.int32),
            jax.ShapeDtypeStruct((B, _P), jnp.float32),
            jax.ShapeDtypeStruct((B, _P), jnp.float32),
            jax.ShapeDtypeStruct((B, _P), jnp.float32),
        ],
        scratch_shapes=[pltpu.VMEM((B, Np), jnp.float32)],
    )(x, y, z, f0)


def _gather_sc(feat2d, inds):
    R, N = feat2d.shape
    B = inds.shape[0]
    C = R // B
    NW = 32
    rpw = R // NW
    mesh = plsc.VectorSubcoreMesh(core_axis_name="c", subcore_axis_name="s")

    @functools.partial(
        pl.kernel,
        out_type=jax.ShapeDtypeStruct((R, _P), jnp.float32),
        mesh=mesh,
        compiler_params=pltpu.CompilerParams(needs_layout_passes=False),
        scratch_types=[
            pltpu.VMEM((_P,), jnp.int32),
            pltpu.VMEM((2, N), jnp.float32),
            pltpu.VMEM((2, N), jnp.float32),
            pltpu.VMEM((rpw, _P), jnp.float32),
            pltpu.SemaphoreType.DMA,
            pltpu.SemaphoreType.DMA,
        ],
    )
    def gather_k(feat_hbm, idx_hbm, out_hbm, idx_v, row_a, row_b,
                 out_acc, sem_a, sem_b):
        cid = lax.axis_index("c")
        sid = lax.axis_index("s")
        wid = sid * 2 + cid
        row0 = wid * rpw
        b = row0 // C
        pltpu.sync_copy(idx_hbm.at[b], idx_v)
        pltpu.async_copy(feat_hbm.at[pl.ds(row0, 2)], row_a, sem_a)

        def pick(buf_ref, j, r):
            for k in range(_P // 16):
                iv = idx_v[pl.ds(k * 16, 16)]
                jv = jnp.full((16,), j, jnp.int32)
                out_acc[r, pl.ds(k * 16, 16)] = plsc.load_gather(
                    buf_ref, [jv, iv])

        def chunk_body(k, _):
            r0 = 4 * k
            pltpu.make_async_copy(feat_hbm.at[pl.ds(row0, 2)],
                                  row_a, sem_a).wait()
            pltpu.async_copy(feat_hbm.at[pl.ds(row0 + r0 + 2, 2)],
                             row_b, sem_b)
            pick(row_a, 0, r0)
            pick(row_a, 1, r0 + 1)
            nxt = jnp.minimum(r0 + 4, rpw - 2)
            pltpu.make_async_copy(feat_hbm.at[pl.ds(row0, 2)],
                                  row_b, sem_b).wait()
            pltpu.async_copy(feat_hbm.at[pl.ds(row0 + nxt, 2)],
                             row_a, sem_a)
            pick(row_b, 0, r0 + 2)
            pick(row_b, 1, r0 + 3)
            return 0

        lax.fori_loop(0, rpw // 4, chunk_body, 0)
        pltpu.make_async_copy(feat_hbm.at[pl.ds(row0, 2)],
                              row_a, sem_a).wait()
        pltpu.sync_copy(out_acc, out_hbm.at[pl.ds(row0, rpw)])

    return gather_k(feat2d, inds)


def kernel(xyz, features):
    B, N, _ = xyz.shape
    C = features.shape[1]
    Np = ((N + _LANES - 1) // _LANES) * _LANES

    feat2d = features.reshape(B * C, N)

    pad = Np - N
    x = jnp.pad(xyz[:, :, 0], ((0, 0), (0, pad)))
    y = jnp.pad(xyz[:, :, 1], ((0, 0), (0, pad)))
    z = jnp.pad(xyz[:, :, 2], ((0, 0), (0, pad)))
    f0 = jax.random.randint(jax.random.key(1), (B,), 0, N,
                            dtype=jnp.int32)[:, None]

    inds, sx, sy, sz = _fps_tc(x, y, z, N, f0)
    new_xyz = jnp.stack([sx, sy, sz], axis=-1)

    new_features = _gather_sc(feat2d, inds).reshape(B, C, _P)
    return (new_xyz, new_features, inds)
